# Initial kernel scaffold; baseline (speedup 1.0000x reference)
import jax
import jax.numpy as jnp
from jax.experimental import pallas as pl


def kernel(token_ids, emb_table):
    raise NotImplementedError("write your pallas kernel here")


# speedup vs baseline: 11.3517x; 11.3517x over previous
import functools

import jax
import jax.numpy as jnp
from jax import lax
from jax.experimental import pallas as pl
from jax.experimental.pallas import tpu as pltpu
from jax.experimental.pallas import tpu_sc as plsc

VOCAB = 100000
D = 128
B = 4096
L = 50

NC = 2
NS = 16
NW = NC * NS
LANES = 16
NCH = D // LANES

SPT = B // NW
IPT = SPT * L
CH_S = 4
CH_I = CH_S * L
NCHUNK = SPT // CH_S

_SCALE = 1.0 / L


def _reduce_chunk(rows_v, out_v, chunk):
    for s in range(CH_S):
        row0 = s * L

        def body(l, accs, _row0=row0):
            return tuple(
                accs[c] + rows_v[_row0 + l, pl.ds(c * LANES, LANES)]
                for c in range(NCH)
            )

        accs = tuple(
            rows_v[row0, pl.ds(c * LANES, LANES)] for c in range(NCH)
        )
        accs = lax.fori_loop(1, L, body, accs)
        orow = chunk * CH_S + s
        for c in range(NCH):
            out_v[orow, pl.ds(c * LANES, LANES)] = accs[c] * jnp.float32(_SCALE)


def kernel(token_ids, emb_table):
    flat_ids = token_ids.reshape(-1).astype(jnp.int32)
    mesh = plsc.VectorSubcoreMesh(core_axis_name="c", subcore_axis_name="s")

    @functools.partial(
        pl.kernel,
        out_type=jax.ShapeDtypeStruct((B, D), jnp.float32),
        mesh=mesh,
        scratch_types=[
            pltpu.VMEM((IPT,), jnp.int32),
            pltpu.VMEM((CH_I, D), jnp.float32),
            pltpu.VMEM((CH_I, D), jnp.float32),
            pltpu.VMEM((SPT, D), jnp.float32),
            pltpu.SemaphoreType.DMA,
            pltpu.SemaphoreType.DMA,
        ],
    )
    def tile_kernel(idx_hbm, table_hbm, out_hbm, idx_v, rows0, rows1, out_v,
                    sem0, sem1):
        wid = lax.axis_index("s") * NC + lax.axis_index("c")
        ibase = wid * IPT
        pltpu.sync_copy(idx_hbm.at[pl.ds(ibase, IPT)], idx_v)
        pltpu.sync_copy(table_hbm.at[idx_v.at[pl.ds(0, CH_I)]], rows0)

        @pl.loop(0, NCHUNK, step=2)
        def _(g):
            cp1 = pltpu.async_copy(
                table_hbm.at[idx_v.at[pl.ds((g + 1) * CH_I, CH_I)]],
                rows1, sem1)
            _reduce_chunk(rows0, out_v, g)
            cp1.wait()

            @pl.when(g + 2 < NCHUNK)
            def _():
                pltpu.async_copy(
                    table_hbm.at[idx_v.at[pl.ds((g + 2) * CH_I, CH_I)]],
                    rows0, sem0)

            _reduce_chunk(rows1, out_v, g + 1)

            @pl.when(g + 2 < NCHUNK)
            def _():
                pltpu.make_async_copy(
                    table_hbm.at[idx_v.at[pl.ds((g + 2) * CH_I, CH_I)]],
                    rows0, sem0).wait()

        pltpu.sync_copy(out_v, out_hbm.at[pl.ds(wid * SPT, SPT)])

    return tile_kernel(flat_ids, emb_table)


# speedup vs baseline: 12.8135x; 1.1288x over previous
import functools

import jax
import jax.numpy as jnp
from jax import lax
from jax.experimental import pallas as pl
from jax.experimental.pallas import tpu as pltpu
from jax.experimental.pallas import tpu_sc as plsc

VOCAB = 100000
D = 128
B = 4096
L = 50

NC = 2
NS = 16
NW = NC * NS
LANES = 16
NCH = D // LANES

SPT = B // NW
IPT = SPT * L
CH_S = 8
CH_I = CH_S * L
NCHUNK = SPT // CH_S

_SCALE = 1.0 / L


def _reduce_chunk(rows_v, out_v, chunk):
    zero = jnp.zeros((LANES,), jnp.float32)
    for s in range(CH_S):
        row0 = s * L
        init = (tuple(zero for _ in range(NCH)), tuple(zero for _ in range(NCH)))

        @plsc.parallel_loop(0, L // 2, carry=init)
        def accs(i, carry, _row0=row0):
            a, b = carry
            ra = _row0 + 2 * i
            a = tuple(
                a[c] + rows_v[ra, pl.ds(c * LANES, LANES)] for c in range(NCH)
            )
            b = tuple(
                b[c] + rows_v[ra + 1, pl.ds(c * LANES, LANES)]
                for c in range(NCH)
            )
            return (a, b)

        a, b = accs
        orow = chunk * CH_S + s
        for c in range(NCH):
            out_v[orow, pl.ds(c * LANES, LANES)] = (a[c] + b[c]) * jnp.float32(
                _SCALE)


def kernel(token_ids, emb_table):
    flat_ids = token_ids.reshape(-1).astype(jnp.int32)
    mesh = plsc.VectorSubcoreMesh(core_axis_name="c", subcore_axis_name="s")

    @functools.partial(
        pl.kernel,
        out_type=jax.ShapeDtypeStruct((B, D), jnp.float32),
        mesh=mesh,
        scratch_types=[
            pltpu.VMEM((IPT,), jnp.int32),
            pltpu.VMEM((CH_I, D), jnp.float32),
            pltpu.VMEM((CH_I, D), jnp.float32),
            pltpu.VMEM((SPT, D), jnp.float32),
            pltpu.SemaphoreType.DMA,
            pltpu.SemaphoreType.DMA,
        ],
    )
    def tile_kernel(idx_hbm, table_hbm, out_hbm, idx_v, rows0, rows1, out_v,
                    sem0, sem1):
        wid = lax.axis_index("s") * NC + lax.axis_index("c")
        ibase = wid * IPT
        pltpu.sync_copy(idx_hbm.at[pl.ds(ibase, IPT)], idx_v)
        pltpu.sync_copy(table_hbm.at[idx_v.at[pl.ds(0, CH_I)]], rows0)

        @pl.loop(0, NCHUNK, step=2)
        def _(g):
            cp1 = pltpu.async_copy(
                table_hbm.at[idx_v.at[pl.ds((g + 1) * CH_I, CH_I)]],
                rows1, sem1)
            _reduce_chunk(rows0, out_v, g)
            cp1.wait()

            @pl.when(g + 2 < NCHUNK)
            def _():
                pltpu.async_copy(
                    table_hbm.at[idx_v.at[pl.ds((g + 2) * CH_I, CH_I)]],
                    rows0, sem0)

            _reduce_chunk(rows1, out_v, g + 1)

            @pl.when(g + 2 < NCHUNK)
            def _():
                pltpu.make_async_copy(
                    table_hbm.at[idx_v.at[pl.ds((g + 2) * CH_I, CH_I)]],
                    rows0, sem0).wait()

        pltpu.sync_copy(out_v, out_hbm.at[pl.ds(wid * SPT, SPT)])

    return tile_kernel(flat_ids, emb_table)


# speedup vs baseline: 15.0906x; 1.1777x over previous
import functools

import jax
import jax.numpy as jnp
from jax import lax
from jax.experimental import pallas as pl
from jax.experimental.pallas import tpu as pltpu
from jax.experimental.pallas import tpu_sc as plsc

VOCAB = 100000
D = 128
B = 4096
L = 50

NC = 2
NS = 16
NW = NC * NS
LANES = 16
NCH = D // LANES

SPT = B // NW
IPT = SPT * L
CH_S = 4
CH_I = CH_S * L
NCHUNK = SPT // CH_S
NBUF = 4

_SCALE = 1.0 / L


def _reduce_chunk(rows_v, out_v, chunk):
    zero = jnp.zeros((LANES,), jnp.float32)
    for s in range(CH_S):
        row0 = s * L
        init = (tuple(zero for _ in range(NCH)), tuple(zero for _ in range(NCH)))

        @plsc.parallel_loop(0, L // 2, carry=init)
        def accs(i, carry, _row0=row0):
            a, b = carry
            ra = _row0 + 2 * i
            a = tuple(
                a[c] + rows_v[ra, pl.ds(c * LANES, LANES)] for c in range(NCH)
            )
            b = tuple(
                b[c] + rows_v[ra + 1, pl.ds(c * LANES, LANES)]
                for c in range(NCH)
            )
            return (a, b)

        a, b = accs
        orow = chunk * CH_S + s
        for c in range(NCH):
            out_v[orow, pl.ds(c * LANES, LANES)] = (a[c] + b[c]) * jnp.float32(
                _SCALE)


def kernel(token_ids, emb_table):
    flat_ids = token_ids.reshape(-1).astype(jnp.int32)
    mesh = plsc.VectorSubcoreMesh(core_axis_name="c", subcore_axis_name="s")

    @functools.partial(
        pl.kernel,
        out_type=jax.ShapeDtypeStruct((B, D), jnp.float32),
        mesh=mesh,
        scratch_types=[
            pltpu.VMEM((IPT,), jnp.int32),
            pltpu.VMEM((NBUF, CH_I, D), jnp.float32),
            pltpu.VMEM((SPT, D), jnp.float32),
        ]
        + [pltpu.SemaphoreType.DMA] * NBUF,
    )
    def tile_kernel(idx_hbm, table_hbm, out_hbm, idx_v, rows_v, out_v, *sems):
        wid = lax.axis_index("s") * NC + lax.axis_index("c")
        ibase = wid * IPT
        pltpu.sync_copy(idx_hbm.at[pl.ds(ibase, IPT)], idx_v)

        def start(chunk, buf):
            pltpu.async_copy(
                table_hbm.at[idx_v.at[pl.ds(chunk * CH_I, CH_I)]],
                rows_v.at[buf], sems[buf])

        def wait(chunk, buf):
            pltpu.make_async_copy(
                table_hbm.at[idx_v.at[pl.ds(chunk * CH_I, CH_I)]],
                rows_v.at[buf], sems[buf]).wait()

        for k in range(NBUF - 1):
            start(k, k)

        @pl.loop(0, NCHUNK, step=NBUF)
        def _(g):
            for k in range(NBUF):
                wait(g + k, k)
                nxt = g + k + (NBUF - 1)

                @pl.when(nxt < NCHUNK)
                def _(_nxt=nxt, _buf=(k + NBUF - 1) % NBUF):
                    start(_nxt, _buf)

                _reduce_chunk(rows_v.at[k], out_v, g + k)

        pltpu.sync_copy(out_v, out_hbm.at[pl.ds(wid * SPT, SPT)])

    return tile_kernel(flat_ids, emb_table)
